# Initial kernel scaffold; baseline (speedup 1.0000x reference)
import jax
import jax.numpy as jnp
from jax.experimental import pallas as pl


def kernel(adj, x, weight, bias):
    raise NotImplementedError("write your pallas kernel here")


# speedup vs baseline: 2.1053x; 2.1053x over previous
import jax
import jax.numpy as jnp
from jax.experimental import pallas as pl
from jax.experimental.pallas import tpu as pltpu


def _round_up(v: int, m: int) -> int:
    return (v + m - 1) // m * m


def _gcn_kernel(adj_ref, x_ref, wt_ref, b_ref, o_ref):
    h = jnp.dot(
        adj_ref[...].astype(jnp.bfloat16),
        x_ref[...],
        preferred_element_type=jnp.float32,
    )
    o_ref[...] = (
        jnp.dot(h.astype(jnp.bfloat16), wt_ref[...], preferred_element_type=jnp.float32)
        + b_ref[...]
    ).astype(o_ref.dtype)


def _gcn(adj, x, wt, b, tm):
    n = adj.shape[0]
    k = adj.shape[1]
    in_f = x.shape[1]
    out_f = wt.shape[1]
    grid = (n // tm,)
    cost = pl.CostEstimate(
        flops=2 * n * k * in_f + 2 * n * in_f * out_f,
        transcendentals=0,
        bytes_accessed=4 * (n * k + n * out_f) + 2 * (k * in_f + in_f * out_f),
    )
    return pl.pallas_call(
        _gcn_kernel,
        out_shape=jax.ShapeDtypeStruct((n, out_f), jnp.float32),
        grid=grid,
        in_specs=[
            pl.BlockSpec((tm, k), lambda i: (i, 0)),
            pl.BlockSpec((k, in_f), lambda i: (0, 0)),
            pl.BlockSpec((in_f, out_f), lambda i: (0, 0)),
            pl.BlockSpec((1, out_f), lambda i: (0, 0)),
        ],
        out_specs=pl.BlockSpec((tm, out_f), lambda i: (i, 0)),
        compiler_params=pltpu.CompilerParams(
            dimension_semantics=("parallel",),
        ),
        cost_estimate=cost,
    )(adj, x, wt, b)


@jax.jit
def _gcn_layer(adj, x, weight, bias):
    n, in_f = x.shape
    out_f = weight.shape[0]

    n_p = _round_up(n, 256)
    in_f_p = _round_up(in_f, 256)
    out_f_p = _round_up(out_f, 256)

    adj_p = adj.astype(jnp.float32)
    x_p = x.astype(jnp.bfloat16)
    wt_p = weight.astype(jnp.bfloat16).T
    b_p = bias.astype(jnp.float32).reshape(1, out_f)
    if (n_p, in_f_p, out_f_p) != (n, in_f, out_f):
        adj_p = jnp.pad(adj_p, ((0, n_p - n), (0, n_p - n)))
        x_p = jnp.pad(x_p, ((0, n_p - n), (0, in_f_p - in_f)))
        wt_p = jnp.pad(wt_p, ((0, in_f_p - in_f), (0, out_f_p - out_f)))
        b_p = jnp.pad(b_p, ((0, 0), (0, out_f_p - out_f)))

    tm = 512 if n_p % 512 == 0 else 256
    out_p = _gcn(adj_p, x_p, wt_p, b_p, tm)
    return out_p[:n, :out_f].astype(x.dtype)


def kernel(adj, x, weight, bias):
    return _gcn_layer(adj, x, weight, bias)


# speedup vs baseline: 2.4902x; 1.1828x over previous
import jax
import jax.numpy as jnp
from jax.experimental import pallas as pl
from jax.experimental.pallas import tpu as pltpu


def _round_up(v: int, m: int) -> int:
    return (v + m - 1) // m * m


def _gcn_kernel(adj_ref, x_ref, w_ref, b_ref, o_ref):
    h = jnp.dot(adj_ref[...], x_ref[...], preferred_element_type=jnp.float32)
    o_ref[...] = (
        jax.lax.dot_general(
            h,
            w_ref[...],
            dimension_numbers=(((1,), (1,)), ((), ())),
            preferred_element_type=jnp.float32,
        )
        + b_ref[...]
    )


def _gcn(adj, x, w, b, tm):
    n, k = adj.shape
    in_f = x.shape[1]
    out_f = w.shape[0]
    grid = (n // tm,)
    cost = pl.CostEstimate(
        flops=2 * n * k * in_f + 2 * n * in_f * out_f,
        transcendentals=0,
        bytes_accessed=4 * (n * k + k * in_f + in_f * out_f + n * out_f),
    )
    return pl.pallas_call(
        _gcn_kernel,
        out_shape=jax.ShapeDtypeStruct((n, out_f), jnp.float32),
        grid=grid,
        in_specs=[
            pl.BlockSpec((tm, k), lambda i: (i, 0)),
            pl.BlockSpec((k, in_f), lambda i: (0, 0)),
            pl.BlockSpec((out_f, in_f), lambda i: (0, 0)),
            pl.BlockSpec((1, out_f), lambda i: (0, 0)),
        ],
        out_specs=pl.BlockSpec((tm, out_f), lambda i: (i, 0)),
        compiler_params=pltpu.CompilerParams(
            dimension_semantics=("parallel",),
        ),
        cost_estimate=cost,
    )(adj, x, w, b)


@jax.jit
def _gcn_layer(adj, x, weight, bias):
    n, in_f = x.shape
    out_f = weight.shape[0]

    n_p = _round_up(n, 256)
    in_f_p = _round_up(in_f, 256)
    out_f_p = _round_up(out_f, 256)

    adj_p = adj.astype(jnp.float32)
    x_p = x.astype(jnp.float32)
    w_p = weight.astype(jnp.float32)
    b_p = bias.astype(jnp.float32).reshape(1, out_f)
    if (n_p, in_f_p, out_f_p) != (n, in_f, out_f):
        adj_p = jnp.pad(adj_p, ((0, n_p - n), (0, n_p - n)))
        x_p = jnp.pad(x_p, ((0, n_p - n), (0, in_f_p - in_f)))
        w_p = jnp.pad(w_p, ((0, out_f_p - out_f), (0, in_f_p - in_f)))
        b_p = jnp.pad(b_p, ((0, 0), (0, out_f_p - out_f)))

    tm = 512 if n_p % 512 == 0 else 256
    out_p = _gcn(adj_p, x_p, w_p, b_p, tm)
    return out_p[:n, :out_f].astype(x.dtype)


def kernel(adj, x, weight, bias):
    return _gcn_layer(adj, x, weight, bias)
